# Initial kernel scaffold; baseline (speedup 1.0000x reference)
import jax
import jax.numpy as jnp
from jax.experimental import pallas as pl


def kernel(pred_ccoords, pred_beta, pred_dist):
    raise NotImplementedError("write your pallas kernel here")


# speedup vs baseline: 39.5563x; 39.5563x over previous
import jax
import jax.numpy as jnp
from jax import lax
from jax.experimental import pallas as pl

_BETA_THRESHOLD = 0.3
_DIST_THRESHOLD = 0.5
_N = 20000
_ROWS = 160
_COLS = 128
_NPAD = _ROWS * _COLS

_NEG_BIG = -3.0e38


def _condense_kernel(cx_ref, cy_ref, cz_ref, beta_ref, dist_ref,
                     assign_ref, alpha_ref, ccx_ref, ccy_ref, ccz_ref):
    cx = cx_ref[:]
    cy = cy_ref[:]
    cz = cz_ref[:]
    beta = beta_ref[:]
    dist = dist_ref[:]

    flat = (lax.broadcasted_iota(jnp.int32, (_ROWS, _COLS), 0) * _COLS
            + lax.broadcasted_iota(jnp.int32, (_ROWS, _COLS), 1))

    assign_ref[:] = jnp.full((_ROWS, _COLS), -1, jnp.int32)
    alpha_ref[:] = jnp.full((_ROWS, _COLS), -1, jnp.int32)
    ccx_ref[:] = jnp.zeros((_ROWS, _COLS), jnp.float32)
    ccy_ref[:] = jnp.zeros((_ROWS, _COLS), jnp.float32)
    ccz_ref[:] = jnp.zeros((_ROWS, _COLS), jnp.float32)

    def argmax_avail(avail):
        m = jnp.max(avail)
        a = jnp.min(jnp.where(avail == m, flat, jnp.int32(2**30)))
        return m, a

    m0, a0 = argmax_avail(beta)

    def body(state):
        k, a, _m = state
        sel = flat == a
        ax = jnp.max(jnp.where(sel, cx, _NEG_BIG))
        ay = jnp.max(jnp.where(sel, cy, _NEG_BIG))
        az = jnp.max(jnp.where(sel, cz, _NEG_BIG))
        ra = jnp.max(jnp.where(sel, dist, _NEG_BIG)) * jnp.float32(_DIST_THRESHOLD)

        dx = cx - ax
        dy = cy - ay
        dz = cz - az
        d = jnp.sqrt(dx * dx + dy * dy + dz * dz)
        assign = assign_ref[:]
        within = (d <= ra) & (assign < 0)
        assign = jnp.where(within, k, assign)
        assign_ref[:] = assign
        alpha_ref[:] = jnp.where(within, a, alpha_ref[:])
        ccx_ref[:] = jnp.where(within, ax, ccx_ref[:])
        ccy_ref[:] = jnp.where(within, ay, ccy_ref[:])
        ccz_ref[:] = jnp.where(within, az, ccz_ref[:])

        avail = jnp.where(assign < 0, beta, jnp.float32(-1.0))
        m2, a2 = argmax_avail(avail)
        return k + jnp.int32(1), a2, m2

    lax.while_loop(lambda s: s[2] > jnp.float32(_BETA_THRESHOLD), body,
                   (jnp.int32(0), a0, m0))


def kernel(pred_ccoords, pred_beta, pred_dist):
    pad = _NPAD - _N
    cx = jnp.pad(pred_ccoords[:, 0], (0, pad), constant_values=1e30)
    cy = jnp.pad(pred_ccoords[:, 1], (0, pad), constant_values=1e30)
    cz = jnp.pad(pred_ccoords[:, 2], (0, pad), constant_values=1e30)
    beta = jnp.pad(pred_beta.reshape(-1), (0, pad), constant_values=-1.0)
    dist = jnp.pad(pred_dist.reshape(-1), (0, pad), constant_values=0.0)

    shape2d = (_ROWS, _COLS)
    args = [a.reshape(shape2d) for a in (cx, cy, cz, beta, dist)]

    out_shape = [
        jax.ShapeDtypeStruct(shape2d, jnp.int32),
        jax.ShapeDtypeStruct(shape2d, jnp.int32),
        jax.ShapeDtypeStruct(shape2d, jnp.float32),
        jax.ShapeDtypeStruct(shape2d, jnp.float32),
        jax.ShapeDtypeStruct(shape2d, jnp.float32),
    ]
    assign2d, alpha2d, ccx, ccy, ccz = pl.pallas_call(
        _condense_kernel,
        out_shape=out_shape,
    )(*args)

    assign = assign2d.reshape(-1)[:_N]
    alpha_idx = alpha2d.reshape(-1)[:_N]
    cond_coords = jnp.stack(
        [ccx.reshape(-1)[:_N], ccy.reshape(-1)[:_N], ccz.reshape(-1)[:_N]],
        axis=-1)
    return assign, alpha_idx, cond_coords
